# Initial kernel scaffold; baseline (speedup 1.0000x reference)
import jax
import jax.numpy as jnp
from jax.experimental import pallas as pl


def kernel(input_ids, weights):
    raise NotImplementedError("write your pallas kernel here")


# speedup vs baseline: 2.1139x; 2.1139x over previous
import functools

import jax
import jax.numpy as jnp
from jax import lax
from jax.experimental import pallas as pl
from jax.experimental.pallas import tpu as pltpu
from jax.experimental.pallas import tpu_sc as plsc

PAD = 1
B = 4
S = 8192
D = 1024
NPOS_ROWS = 8194

NC = 2
NS = 16
L = 16

NW = NC * NS
TOK_PER_W = (B * S) // NW
W_PER_ROW = S // TOK_PER_W
CHUNK = 64
NCHUNKS = TOK_PER_W // CHUNK


def _sc_body(ids_hbm, w_hbm, out_hbm, ids_v, msk_v, csum_v, idx_v,
             stage_v, tot_v, rows_v, tot_sh, gsem):
    cid = lax.axis_index("c")
    sid = lax.axis_index("s")
    row = 2 * cid + sid // W_PER_ROW
    slot = sid % W_PER_ROW
    tbase = row * S + slot * TOK_PER_W

    pltpu.sync_copy(ids_hbm.at[pl.ds(tbase, TOK_PER_W)], ids_v)

    def cs_body(i, carry):
        v = ids_v[pl.ds(i * L, L)]
        m = jnp.where(v != PAD, 1, 0).astype(jnp.int32)
        c = plsc.cumsum(m) + carry
        msk_v[pl.ds(i * L, L)] = m
        csum_v[pl.ds(i * L, L)] = c
        return jnp.max(c)

    total = lax.fori_loop(0, TOK_PER_W // L, cs_body, jnp.int32(0))

    stage_v[...] = jnp.full((L,), total, jnp.int32)
    pltpu.sync_copy(stage_v, tot_sh.at[pl.ds(sid * L, L)])
    plsc.subcore_barrier()
    pltpu.sync_copy(tot_sh, tot_v)

    rstart = (sid // W_PER_ROW) * W_PER_ROW
    offset = jnp.int32(0)
    for jj in range(W_PER_ROW):
        j = rstart + jj
        t = jnp.max(tot_v[pl.ds(j * L, L)])
        offset = offset + jnp.where(j < sid, t, 0).astype(jnp.int32)

    def idx_body(i, _):
        c = csum_v[pl.ds(i * L, L)]
        m = msk_v[pl.ds(i * L, L)]
        idx_v[pl.ds(i * L, L)] = (c + offset) * m + PAD
        return 0

    lax.fori_loop(0, TOK_PER_W // L, idx_body, 0)

    def g_body(k, _):
        cp = pltpu.async_copy(
            w_hbm.at[idx_v.at[pl.ds(k * CHUNK, CHUNK)]], rows_v, gsem)
        cp.wait()
        pltpu.sync_copy(rows_v, out_hbm.at[pl.ds(tbase + k * CHUNK, CHUNK)])
        return 0

    lax.fori_loop(0, NCHUNKS, g_body, 0)


@jax.jit
def _sc_embed(ids_flat, weights):
    mesh = plsc.VectorSubcoreMesh(
        core_axis_name="c", subcore_axis_name="s",
        num_cores=NC, num_subcores=NS)
    f = pl.kernel(
        _sc_body,
        out_type=jax.ShapeDtypeStruct((B * S, D), jnp.float32),
        mesh=mesh,
        compiler_params=pltpu.CompilerParams(needs_layout_passes=False),
        scratch_types=[
            pltpu.VMEM((TOK_PER_W,), jnp.int32),
            pltpu.VMEM((TOK_PER_W,), jnp.int32),
            pltpu.VMEM((TOK_PER_W,), jnp.int32),
            pltpu.VMEM((TOK_PER_W,), jnp.int32),
            pltpu.VMEM((L,), jnp.int32),
            pltpu.VMEM((NS * L,), jnp.int32),
            pltpu.VMEM((CHUNK, D), jnp.float32),
            pltpu.VMEM_SHARED((NS * L,), jnp.int32),
            pltpu.SemaphoreType.DMA,
        ],
    )
    return f(ids_flat, weights)


def kernel(input_ids, weights):
    out = _sc_embed(input_ids.reshape(-1), weights)
    return out.reshape(B, S, D)


# speedup vs baseline: 2.3196x; 1.0973x over previous
import jax
import jax.numpy as jnp
from jax import lax
from jax.experimental import pallas as pl
from jax.experimental.pallas import tpu as pltpu
from jax.experimental.pallas import tpu_sc as plsc

PAD = 1
B = 4
S = 8192
D = 1024

NC = 2
NS = 16
L = 16

NW = NC * NS
TOK_PER_W = (B * S) // NW
W_PER_ROW = S // TOK_PER_W
CHUNK = 32
NBUF = 2
NCHUNKS = TOK_PER_W // CHUNK
NSTEPS = NCHUNKS // NBUF


def _sc_body(ids_hbm, w_hbm, out_hbm, ids_v, stage_v, tot_v,
             rows_v, tot_sh, g0, g1, s0, s1):
    gsem = [g0, g1]
    ssem = [s0, s1]
    cid = lax.axis_index("c")
    sid = lax.axis_index("s")
    row = 2 * cid + sid // W_PER_ROW
    slot = sid % W_PER_ROW
    tbase = row * S + slot * TOK_PER_W

    pltpu.sync_copy(ids_hbm.at[pl.ds(tbase, TOK_PER_W)], ids_v)

    def cs_body(i, carry):
        v = ids_v[pl.ds(i * L, L)]
        m = jnp.where(v != PAD, 1, 0).astype(jnp.int32)
        c = plsc.cumsum(m) + carry
        ids_v[pl.ds(i * L, L)] = c * m
        return jnp.max(c)

    total = lax.fori_loop(0, TOK_PER_W // L, cs_body, jnp.int32(0))

    stage_v[...] = jnp.full((L,), total, jnp.int32)
    pltpu.sync_copy(stage_v, tot_sh.at[pl.ds(sid * L, L)])
    plsc.subcore_barrier()
    pltpu.sync_copy(tot_sh, tot_v)

    rstart = (sid // W_PER_ROW) * W_PER_ROW
    offset = jnp.int32(0)
    for jj in range(W_PER_ROW):
        j = rstart + jj
        t = jnp.max(tot_v[pl.ds(j * L, L)])
        offset = offset + jnp.where(j < sid, t, 0).astype(jnp.int32)

    def idx_body(i, _):
        e = ids_v[pl.ds(i * L, L)]
        m = jnp.where(e > 0, 1, 0).astype(jnp.int32)
        ids_v[pl.ds(i * L, L)] = e + offset * m + PAD
        return 0

    lax.fori_loop(0, TOK_PER_W // L, idx_body, 0)

    def gather_start(k, b):
        pltpu.async_copy(
            w_hbm.at[ids_v.at[pl.ds(k * CHUNK, CHUNK)]], rows_v.at[b],
            gsem[b])

    def gather_wait(b):
        pltpu.make_async_copy(
            w_hbm.at[ids_v.at[pl.ds(0, CHUNK)]], rows_v.at[b],
            gsem[b]).wait()

    def scatter_start(k, b):
        pltpu.async_copy(
            rows_v.at[b], out_hbm.at[pl.ds(tbase + k * CHUNK, CHUNK)],
            ssem[b])

    def scatter_wait(b):
        pltpu.make_async_copy(
            rows_v.at[b], out_hbm.at[pl.ds(0, CHUNK)], ssem[b]).wait()

    for b in range(NBUF):
        gather_start(b, b)

    def pipe_body(step, _):
        for b in range(NBUF):
            k = step * NBUF + b
            gather_wait(b)
            scatter_start(k, b)
            scatter_wait(b)
            gather_start(k + NBUF, b)
        return 0

    lax.fori_loop(0, NSTEPS - 1, pipe_body, 0)

    for b in range(NBUF):
        k = (NSTEPS - 1) * NBUF + b
        gather_wait(b)
        scatter_start(k, b)
        scatter_wait(b)


@jax.jit
def _sc_embed(ids_flat, weights):
    mesh = plsc.VectorSubcoreMesh(
        core_axis_name="c", subcore_axis_name="s",
        num_cores=NC, num_subcores=NS)
    f = pl.kernel(
        _sc_body,
        out_type=jax.ShapeDtypeStruct((B * S, D), jnp.float32),
        mesh=mesh,
        compiler_params=pltpu.CompilerParams(needs_layout_passes=False),
        scratch_types=[
            pltpu.VMEM((TOK_PER_W,), jnp.int32),
            pltpu.VMEM((L,), jnp.int32),
            pltpu.VMEM((NS * L,), jnp.int32),
            pltpu.VMEM((NBUF, CHUNK, D), jnp.float32),
            pltpu.VMEM_SHARED((NS * L,), jnp.int32),
            pltpu.SemaphoreType.DMA,
            pltpu.SemaphoreType.DMA,
            pltpu.SemaphoreType.DMA,
            pltpu.SemaphoreType.DMA,
        ],
    )
    return f(ids_flat, weights)


def kernel(input_ids, weights):
    out = _sc_embed(input_ids.reshape(-1), weights)
    return out.reshape(B, S, D)
